# Initial kernel scaffold; baseline (speedup 1.0000x reference)
import jax
import jax.numpy as jnp
from jax.experimental import pallas as pl


def kernel(x, edge_index, params):
    raise NotImplementedError("write your pallas kernel here")


# speedup vs baseline: 1.1609x; 1.1609x over previous
import functools

import jax
import jax.numpy as jnp
import numpy as np
from jax import lax
from jax.experimental import pallas as pl
from jax.experimental.pallas import tpu as pltpu
from jax.experimental.pallas import tpu_sc as plsc

NC = 2
NS = 16
NW = NC * NS
L = 16
PAD = 128
CH = 128


@functools.lru_cache(maxsize=None)
def _make_segsum(F, Npad, n_chunks):
  rows_pt = Npad // NS
  mesh = plsc.VectorSubcoreMesh(
      core_axis_name="c", subcore_axis_name="s",
      num_cores=NC, num_subcores=NS)

  @functools.partial(
      pl.kernel,
      out_type=jax.ShapeDtypeStruct((NC, Npad, F), jnp.float32),
      mesh=mesh,
      scratch_types=[
          pltpu.VMEM((n_chunks, CH), jnp.int32),
          pltpu.VMEM((n_chunks, CH), jnp.int32),
          pltpu.VMEM((CH, F), jnp.float32),
          pltpu.VMEM_SHARED((Npad, F), jnp.float32),
          pltpu.SemaphoreType.DMA,
      ],
      compiler_params=pltpu.CompilerParams(use_tc_tiling_on_sc=False),
  )
  def segsum(table, src, dst, zeros, out, src_v, dst_v, rows_v, acc, sem):
    c = lax.axis_index("c")
    s = lax.axis_index("s")
    wid = c * NS + s
    pltpu.sync_copy(zeros.at[pl.ds(s * rows_pt, rows_pt)],
                    acc.at[pl.ds(s * rows_pt, rows_pt)])
    pltpu.sync_copy(src.at[wid], src_v)
    pltpu.sync_copy(dst.at[wid], dst_v)
    plsc.subcore_barrier()

    def body(i, carry):
      pltpu.async_copy(table.at[src_v.at[i]], rows_v, sem).wait()
      pltpu.sync_copy(rows_v, acc.at[dst_v.at[i]], add=True)
      return carry

    lax.fori_loop(0, n_chunks, body, 0)
    plsc.subcore_barrier()
    pltpu.sync_copy(acc.at[pl.ds(s * rows_pt, rows_pt)],
                    out.at[c, pl.ds(s * rows_pt, rows_pt)])

  return segsum


def _segsum(table, src3, dst3, Npad):
  F = table.shape[1]
  n_chunks = src3.shape[1]
  k = _make_segsum(F, Npad, n_chunks)
  zeros = jnp.zeros((Npad, F), jnp.float32)
  parts = k(table, src3, dst3, zeros)
  return parts[0] + parts[1]


def _pack_edges(src, dst, trash):
  e0 = src.shape[0]
  n_chunks = -(-e0 // (NW * CH))
  pad = NW * n_chunks * CH - e0
  if pad:
    src = jnp.concatenate([src, jnp.zeros((pad,), jnp.int32)])
    dst = jnp.concatenate([dst, jnp.full((pad,), trash, jnp.int32)])
  return src.reshape(NW, n_chunks, CH), dst.reshape(NW, n_chunks, CH)


def _count(dst3, n):
  ones_tab = jnp.ones((16, L), jnp.float32)
  src3 = jnp.zeros_like(dst3)
  return _segsum(ones_tab, src3, dst3, n + PAD)[:n, 0]


def _pad16(h):
  f = h.shape[1]
  r = (-f) % L
  return jnp.pad(h, ((0, 0), (0, r))) if r else h


def _gcn(h, src3, dsteff3, dinv, b, n):
  f = h.shape[1]
  hp = _pad16(h * dinv[:, None])
  msg = _segsum(hp, src3, dsteff3, n + PAD)[:n, :f]
  return dinv[:, None] * msg + h * (2.0 * dinv * dinv)[:, None] + b


def _bn(x, g, b, eps=1e-5):
  m = jnp.mean(x, axis=0)
  v = jnp.mean((x - m) ** 2, axis=0)
  return (x - m) / jnp.sqrt(v + eps) * g + b


def _pool(xh, src, dst, ew, p):
  n = xh.shape[0]
  k = n // 2
  score = jnp.tanh((xh @ p) / jnp.linalg.norm(p))
  topv, perm = lax.top_k(score, k)
  gs, gd = _pack_edges(perm.astype(jnp.int32),
                       jnp.arange(k, dtype=jnp.int32), k)
  xn = _segsum(_pad16(xh), gs, gd, k + PAD)[:k, :xh.shape[1]] * topv[:, None]
  cluster = (jnp.full((n,), -1, jnp.int32)
             .at[perm].set(jnp.arange(k, dtype=jnp.int32)))
  s = cluster[src]
  d = cluster[dst]
  valid = (s >= 0) & (d >= 0)
  s = jnp.where(valid, s, 0)
  d = jnp.where(valid, d, 0)
  ewn = jnp.where(valid, ew, 0.0)
  return xn, s, d, ewn, perm


def _unpool(h_small, perm, n_big):
  k = h_small.shape[0]
  f = h_small.shape[1]
  gs, gd = _pack_edges(jnp.arange(k, dtype=jnp.int32),
                       perm.astype(jnp.int32), n_big)
  return _segsum(_pad16(h_small), gs, gd, n_big + PAD)[:n_big, :f]


def _level(h_in, src, dst, ew, n, W1, b1, g1, bt1, W2, b2, g2, bt2):
  elu = jax.nn.elu
  dsteff = jnp.where(ew > 0, dst, n)
  s3, d3 = _pack_edges(src, dsteff, n)
  deg = _count(d3, n) + 2.0
  dinv = 1.0 / jnp.sqrt(deg)
  h = _bn(elu(_gcn(h_in @ W1, s3, d3, dinv, b1, n)), g1, bt1)
  h = _bn(elu(_gcn(h @ W2, s3, d3, dinv, b2, n)), g2, bt2)
  return h, s3, d3, dinv


def _proj(h, s3, d3, dinv, n, W, b, g, bt):
  return _bn(jax.nn.elu(_gcn(h @ W, s3, d3, dinv, b, n)), g, bt)


def kernel(x, edge_index, params):
  P = params
  n0 = x.shape[0]
  e = edge_index.shape[1]
  src0, dst0 = edge_index[0], edge_index[1]
  ew0 = jnp.ones((e,), jnp.float32)

  hA, sA, dA, dinvA = _level(x, src0, dst0, ew0, n0,
                             P['W1'], P['b1'], P['g1'], P['bt1'],
                             P['W2'], P['b2'], P['g2'], P['bt2'])
  hB0, srcB, dstB, ewB, permB = _pool(hA, src0, dst0, ew0, P['p1'])
  nB = hB0.shape[0]
  hB, sB, dB, dinvB = _level(hB0, srcB, dstB, ewB, nB,
                             P['W3'], P['b3'], P['g3'], P['bt3'],
                             P['W4'], P['b4'], P['g4'], P['bt4'])
  hC0, srcC, dstC, ewC, permC = _pool(hB, srcB, dstB, ewB, P['p2'])
  nC = hC0.shape[0]
  hC, sC, dC, dinvC = _level(hC0, srcC, dstC, ewC, nC,
                             P['W5'], P['b5'], P['g5'], P['bt5'],
                             P['W6'], P['b6'], P['g6'], P['bt6'])
  hD0, srcD, dstD, ewD, permD = _pool(hC, srcC, dstC, ewC, P['p3'])
  nD = hD0.shape[0]
  hD, sD, dD, dinvD = _level(hD0, srcD, dstD, ewD, nD,
                             P['W7'], P['b7'], P['g7'], P['bt7'],
                             P['W8'], P['b8'], P['g8'], P['bt8'])

  HA = _proj(hA, sA, dA, dinvA, n0, P['WPa'], P['bPa'], P['gP'], P['btP'])
  HB = _proj(hB, sB, dB, dinvB, nB, P['WPb'], P['bPb'], P['gP'], P['btP'])
  HC = _proj(hC, sC, dC, dinvC, nC, P['WPc'], P['bPc'], P['gP'], P['btP'])
  HD = _proj(hD, sD, dD, dinvD, nD, P['WPd'], P['bPd'], P['gP'], P['btP'])

  HCD = _unpool(HD, permD, nC) + HC
  HBCD = _unpool(HCD, permC, nB) + HB
  HABCD = _unpool(HBCD, permB, n0) + HA
  return _bn(HABCD, P['gP'], P['btP'])
